# Initial kernel scaffold; baseline (speedup 1.0000x reference)
import jax
import jax.numpy as jnp
from jax.experimental import pallas as pl


def kernel(x, W, gamma, beta, temperature):
    raise NotImplementedError("write your pallas kernel here")


# speedup vs baseline: 4.1878x; 4.1878x over previous
import jax
import jax.numpy as jnp
from jax.experimental import pallas as pl
from jax.experimental.pallas import tpu as pltpu

_INPUT_DIM = 4096
_NUM_EXPERTS = 64
_TOP_K = 8
_BLOCK_M = 512


def _router_kernel(x_ref, wt_ref, gamma_ref, beta_ref, temp_ref,
                   rw_ref, disp_ref, loss_ref,
                   load_acc, z_acc):
    i = pl.program_id(0)
    nsteps = pl.num_programs(0)

    logits = jnp.dot(x_ref[...], wt_ref[...], preferred_element_type=jnp.float32)

    mu = jnp.mean(logits, axis=-1, keepdims=True)
    var = jnp.mean((logits - mu) ** 2, axis=-1, keepdims=True)
    h = (logits - mu) * jax.lax.rsqrt(var + 1e-5) * gamma_ref[...] + beta_ref[...]
    h = h / (jnp.abs(temp_ref[...]) + 1e-6)

    zsum = jnp.sum(h * h)

    hmax = jnp.max(h, axis=-1, keepdims=True)
    e = jnp.exp(h - hmax)
    w = e / jnp.sum(e, axis=-1, keepdims=True)
    rw_ref[...] = w

    col = jnp.sum(w, axis=0, keepdims=True)

    @pl.when(i == 0)
    def _init():
        load_acc[...] = jnp.zeros_like(load_acc)
        z_acc[0, 0] = 0.0

    load_acc[...] += col
    z_acc[0, 0] += zsum

    lanes = jax.lax.broadcasted_iota(jnp.int32, w.shape, 1)
    wc = w
    disp = jnp.zeros_like(w)
    ksum = jnp.zeros((w.shape[0], 1), jnp.float32)
    for _ in range(_TOP_K):
        m = jnp.max(wc, axis=-1, keepdims=True)
        ismax = wc == m
        first = jnp.min(jnp.where(ismax, lanes, _NUM_EXPERTS), axis=-1,
                        keepdims=True)
        chosen = lanes == first
        disp = jnp.where(chosen, w, disp)
        ksum = ksum + m
        wc = jnp.where(chosen, -jnp.inf, wc)
    disp_ref[...] = disp / (ksum + 1e-6)

    @pl.when(i == nsteps - 1)
    def _finalize():
        n_rows = nsteps * _BLOCK_M
        actual = load_acc[...] / n_rows
        ideal = 1.0 / _NUM_EXPERTS
        lb = jnp.sum(ideal * (jnp.log(ideal) - jnp.log(actual))) / _NUM_EXPERTS
        z = z_acc[0, 0] / (n_rows * _NUM_EXPERTS)
        loss_ref[...] = jnp.reshape(0.01 * z + 0.01 * lb, (1, 1))


@jax.jit
def kernel(x, W, gamma, beta, temperature):
    B, S, D = x.shape
    flat = x.reshape(-1, D)
    N = flat.shape[0]
    wt = W.T
    grid = N // _BLOCK_M

    rw, disp, loss = pl.pallas_call(
        _router_kernel,
        grid=(grid,),
        in_specs=[
            pl.BlockSpec((_BLOCK_M, D), lambda i: (i, 0)),
            pl.BlockSpec((D, _NUM_EXPERTS), lambda i: (0, 0)),
            pl.BlockSpec((1, _NUM_EXPERTS), lambda i: (0, 0)),
            pl.BlockSpec((1, _NUM_EXPERTS), lambda i: (0, 0)),
            pl.BlockSpec((1, 1), lambda i: (0, 0)),
        ],
        out_specs=[
            pl.BlockSpec((_BLOCK_M, _NUM_EXPERTS), lambda i: (i, 0)),
            pl.BlockSpec((_BLOCK_M, _NUM_EXPERTS), lambda i: (i, 0)),
            pl.BlockSpec((1, 1), lambda i: (0, 0)),
        ],
        out_shape=[
            jax.ShapeDtypeStruct((N, _NUM_EXPERTS), jnp.float32),
            jax.ShapeDtypeStruct((N, _NUM_EXPERTS), jnp.float32),
            jax.ShapeDtypeStruct((1, 1), jnp.float32),
        ],
        scratch_shapes=[
            pltpu.VMEM((1, _NUM_EXPERTS), jnp.float32),
            pltpu.SMEM((1, 1), jnp.float32),
        ],
    )(flat, wt, gamma.reshape(1, -1), beta.reshape(1, -1),
      temperature.reshape(1, 1))

    return (jax.lax.stop_gradient(rw),
            disp.reshape(B, S, _NUM_EXPERTS),
            loss[0, 0])


# speedup vs baseline: 4.7755x; 1.1403x over previous
import jax
import jax.numpy as jnp
from jax.experimental import pallas as pl
from jax.experimental.pallas import tpu as pltpu

_INPUT_DIM = 4096
_NUM_EXPERTS = 64
_TOP_K = 8
_BLOCK_M = 1024


def _router_kernel(x_ref, wt_ref, gamma_ref, beta_ref, temp_ref,
                   rw_ref, disp_ref, loss_ref,
                   load_acc, z_acc):
    i = pl.program_id(0)
    nsteps = pl.num_programs(0)

    logits = jnp.dot(x_ref[...], wt_ref[...], preferred_element_type=jnp.float32)

    mu = jnp.mean(logits, axis=-1, keepdims=True)
    var = jnp.mean((logits - mu) ** 2, axis=-1, keepdims=True)
    h = (logits - mu) * jax.lax.rsqrt(var + 1e-5) * gamma_ref[...] + beta_ref[...]
    h = h / (jnp.abs(temp_ref[...]) + 1e-6)

    zsum = jnp.sum(h * h)

    hmax = jnp.max(h, axis=-1, keepdims=True)
    e = jnp.exp(h - hmax)
    w = e / jnp.sum(e, axis=-1, keepdims=True)
    rw_ref[...] = w

    col = jnp.sum(w, axis=0, keepdims=True)

    @pl.when(i == 0)
    def _init():
        load_acc[...] = jnp.zeros_like(load_acc)
        z_acc[0, 0] = 0.0

    load_acc[...] += col
    z_acc[0, 0] += zsum

    lanes = jax.lax.broadcasted_iota(jnp.int32, w.shape, 1)
    wc = w
    disp = jnp.zeros_like(w)
    ksum = jnp.zeros((w.shape[0], 1), jnp.float32)
    for _ in range(_TOP_K):
        m = jnp.max(wc, axis=-1, keepdims=True)
        ismax = wc == m
        first = jnp.min(jnp.where(ismax, lanes, _NUM_EXPERTS), axis=-1,
                        keepdims=True)
        chosen = lanes == first
        disp = jnp.where(chosen, w, disp)
        ksum = ksum + m
        wc = jnp.where(chosen, -jnp.inf, wc)
    disp_ref[...] = disp / (ksum + 1e-6)

    @pl.when(i == nsteps - 1)
    def _finalize():
        n_rows = nsteps * _BLOCK_M
        actual = load_acc[...] / n_rows
        ideal = 1.0 / _NUM_EXPERTS
        lb = jnp.sum(ideal * (jnp.log(ideal) - jnp.log(actual))) / _NUM_EXPERTS
        z = z_acc[0, 0] / (n_rows * _NUM_EXPERTS)
        loss_ref[...] = jnp.reshape(0.01 * z + 0.01 * lb, (1, 1))


@jax.jit
def kernel(x, W, gamma, beta, temperature):
    B, S, D = x.shape
    flat = x.reshape(-1, D)
    N = flat.shape[0]
    wt = W.T
    grid = N // _BLOCK_M

    rw, disp, loss = pl.pallas_call(
        _router_kernel,
        grid=(grid,),
        in_specs=[
            pl.BlockSpec((_BLOCK_M, D), lambda i: (i, 0)),
            pl.BlockSpec((D, _NUM_EXPERTS), lambda i: (0, 0)),
            pl.BlockSpec((1, _NUM_EXPERTS), lambda i: (0, 0)),
            pl.BlockSpec((1, _NUM_EXPERTS), lambda i: (0, 0)),
            pl.BlockSpec((1, 1), lambda i: (0, 0)),
        ],
        out_specs=[
            pl.BlockSpec((_BLOCK_M, _NUM_EXPERTS), lambda i: (i, 0)),
            pl.BlockSpec((_BLOCK_M, _NUM_EXPERTS), lambda i: (i, 0)),
            pl.BlockSpec((1, 1), lambda i: (0, 0)),
        ],
        out_shape=[
            jax.ShapeDtypeStruct((N, _NUM_EXPERTS), jnp.float32),
            jax.ShapeDtypeStruct((N, _NUM_EXPERTS), jnp.float32),
            jax.ShapeDtypeStruct((1, 1), jnp.float32),
        ],
        scratch_shapes=[
            pltpu.VMEM((1, _NUM_EXPERTS), jnp.float32),
            pltpu.SMEM((1, 1), jnp.float32),
        ],
    )(flat, wt, gamma.reshape(1, -1), beta.reshape(1, -1),
      temperature.reshape(1, 1))

    return (jax.lax.stop_gradient(rw),
            disp.reshape(B, S, _NUM_EXPERTS),
            loss[0, 0])
